# Initial kernel scaffold; baseline (speedup 1.0000x reference)
import jax
import jax.numpy as jnp
from jax.experimental import pallas as pl


def kernel(x, edge_index, edge_type_in, edge_attr, W_msg, rel_emb, W_rel, att_vec, bias):
    raise NotImplementedError("write your pallas kernel here")


# speedup vs baseline: 19.8886x; 19.8886x over previous
import jax
import jax.numpy as jnp
from jax import lax
from jax.experimental import pallas as pl
from jax.experimental.pallas import tpu as pltpu
from jax.experimental.pallas import tpu_sc as plsc

N = 10000
E = 320000
HID = 128
NUM_RELS = 16
CONF_LOG_WEIGHT = 0.5

NC, NS, L = 2, 16, 16
NPAD = 10240
CPT = NPAD // NS
EPS = E // NS
SUB = 2000
NSUB = EPS // SUB
EPW = E // (NC * NS)
K = 80
NCH = EPW // K
RPB = HID // L


def _prep_body(x_ref, wm_ref, re_ref, wr_ref, av_ref, cf_ref,
               h_ref, sd_ref, ss_ref, rs_ref, cl_ref):
    x = x_ref[...]
    h = lax.dot_general(x, wm_ref[...], (((1,), (1,)), ((), ())),
                        preferred_element_type=jnp.float32)
    h_ref[...] = h
    av = av_ref[...]
    sd_ref[...] = jnp.sum(h * av[0:HID][None, :], axis=1, keepdims=True)
    ss_ref[...] = jnp.sum(h * av[HID:2 * HID][None, :], axis=1, keepdims=True)
    rp = lax.dot_general(re_ref[...], wr_ref[...], (((1,), (1,)), ((), ())),
                         preferred_element_type=jnp.float32)
    rs_ref[...] = jnp.sum(rp * av[2 * HID:][None, :], axis=1, keepdims=True)
    cl_ref[...] = CONF_LOG_WEIGHT * jnp.log(jnp.maximum(cf_ref[...], 1e-6))


def _fin_body(p_ref, b_ref, o_ref):
    o_ref[...] = p_ref[0, :N, :] + p_ref[1, :N, :] + b_ref[...][None, :]


def _alpha_body(src_hbm, dst_hbm, typ_hbm, clog_hbm, sd_hbm, ss_hbm, rsc_hbm,
                al_hbm,
                sd_v, ss_v, rsc_v, e_v, dst_v, sb, tb, cb,
                den_v, mrg_v, tsl_v, mx_v, gst_v,
                gstat, dstg):
    cid = lax.axis_index("c")
    tid = lax.axis_index("s")
    zeros16 = jnp.zeros((L,), jnp.float32)

    pltpu.sync_copy(sd_hbm, sd_v)
    pltpu.sync_copy(ss_hbm, ss_v)
    pltpu.sync_copy(rsc_hbm, rsc_v)

    def _zden(j, _):
        den_v[pl.ds(pl.multiple_of(j * L, L), L)] = zeros16
        return 0
    lax.fori_loop(0, NPAD // L, _zden, 0)

    stats_base = tid * EPS
    mx = jnp.full((L,), -1e30, jnp.float32)
    for sub in range(NSUB):
        off = stats_base + sub * SUB
        pltpu.sync_copy(src_hbm.at[pl.ds(off, SUB)], sb)
        pltpu.sync_copy(typ_hbm.at[pl.ds(off, SUB)], tb)
        pltpu.sync_copy(clog_hbm.at[pl.ds(off, SUB)], cb)
        pltpu.sync_copy(dst_hbm.at[pl.ds(off, SUB)],
                        dst_v.at[pl.ds(sub * SUB, SUB)])

        def _grp(g, mxc, sub=sub):
            o = pl.multiple_of(g * L, L)
            d16 = dst_v[pl.ds(sub * SUB + o, L)]
            s16 = sb[pl.ds(o, L)]
            t16 = tb[pl.ds(o, L)]
            c16 = cb[pl.ds(o, L)]
            sd = plsc.load_gather(sd_v, [d16])
            ssg = plsc.load_gather(ss_v, [s16])
            tt = jnp.clip(t16, 0, NUM_RELS - 1)
            rr = plsc.load_gather(rsc_v, [tt])
            a = sd + ssg + rr
            a = jnp.where(a >= 0, a, 0.2 * a)
            e16 = a + c16
            e_v[pl.ds(sub * SUB + o, L)] = e16
            return jnp.maximum(mxc, e16)
        mx = lax.fori_loop(0, SUB // L, _grp, mx)

    mx_v[...] = mx
    pltpu.sync_copy(mx_v, gstat.at[tid])
    plsc.subcore_barrier()
    pltpu.sync_copy(gstat, gst_v)
    m16 = gst_v[0, :]
    for j in range(1, NS):
        m16 = jnp.maximum(m16, gst_v[j, :])
    gmax = jnp.max(m16)

    def _dgrp(g, _):
        o = pl.multiple_of(g * L, L)
        e16 = e_v[pl.ds(o, L)]
        d16 = dst_v[pl.ds(o, L)]
        ex = jnp.exp(e16 - gmax)
        plsc.addupdate_scatter(den_v, [d16], ex)
        return 0
    lax.fori_loop(0, EPS // L, _dgrp, 0)

    pltpu.sync_copy(den_v, dstg.at[tid])
    plsc.subcore_barrier()
    colbase = tid * CPT
    pltpu.sync_copy(dstg.at[0, pl.ds(colbase, CPT)], mrg_v)
    for j in range(1, NS):
        pltpu.sync_copy(dstg.at[j, pl.ds(colbase, CPT)], tsl_v)

        def _macc(q, _):
            o = pl.multiple_of(q * L, L)
            mrg_v[pl.ds(o, L)] = mrg_v[pl.ds(o, L)] + tsl_v[pl.ds(o, L)]
            return 0
        lax.fori_loop(0, CPT // L, _macc, 0)
    pltpu.sync_copy(mrg_v, dstg.at[0, pl.ds(colbase, CPT)])
    plsc.subcore_barrier()
    pltpu.sync_copy(dstg.at[0], den_v)

    msg_base = (tid * NC + cid) * EPW
    loc = cid * EPW
    for blk in range(EPW // SUB):
        def _agrp(g, _, blk=blk):
            o = pl.multiple_of(g * L, L)
            e16 = e_v[pl.ds(loc + blk * SUB + o, L)]
            d16 = dst_v[pl.ds(loc + blk * SUB + o, L)]
            den16 = plsc.load_gather(den_v, [d16])
            cb[pl.ds(o, L)] = jnp.exp(e16 - gmax) / (den16 + 1e-16)
            return 0
        lax.fori_loop(0, SUB // L, _agrp, 0)
        pltpu.sync_copy(cb, al_hbm.at[pl.ds(msg_base + blk * SUB, SUB)])


def _msg_body(src_hbm, dst_hbm, al_hbm, h_hbm, out_hbm,
              rb0, rb1, rb2, si0, si1, si2, di0, di1, di2, ab0, ab1, ab2,
              gs0, gs1, gs2, sc0, sc1, sc2,
              acc):
    cid = lax.axis_index("c")
    tid = lax.axis_index("s")
    rbs = (rb0, rb1, rb2)
    sis = (si0, si1, si2)
    dis = (di0, di1, di2)
    abs_ = (ab0, ab1, ab2)
    gsem = (gs0, gs1, gs2)
    ssem = (sc0, sc1, sc2)
    zeros16 = jnp.zeros((L,), jnp.float32)

    def _zrow(j, _):
        for r in range(RPB):
            rb0[j, pl.ds(r * L, L)] = zeros16
        return 0
    lax.fori_loop(0, K, _zrow, 0)
    row0 = tid * CPT
    for k in range(CPT // K):
        pltpu.sync_copy(rb0, acc.at[pl.ds(row0 + k * K, K)])
    plsc.subcore_barrier()

    msg_base = (tid * NC + cid) * EPW

    def issue_gather(ci, b):
        goff = pl.multiple_of(msg_base + ci * K, 8)
        pltpu.sync_copy(src_hbm.at[pl.ds(goff, K)], sis[b])
        pltpu.sync_copy(dst_hbm.at[pl.ds(goff, K)], dis[b])
        pltpu.sync_copy(al_hbm.at[pl.ds(goff, K)], abs_[b])
        pltpu.async_copy(h_hbm.at[sis[b]], rbs[b], gsem[b])

    def wait_gather(b):
        pltpu.make_async_copy(h_hbm.at[sis[b]], rbs[b], gsem[b]).wait()

    def compute_scale(b):
        def _srow(j, _):
            s16 = plsc.load_gather(abs_[b], [jnp.full((L,), j, jnp.int32)])
            for r in range(RPB):
                rbs[b][j, pl.ds(r * L, L)] = rbs[b][j, pl.ds(r * L, L)] * s16
            return 0
        lax.fori_loop(0, K, _srow, 0)

    def issue_scatter(b):
        pltpu.async_copy(rbs[b], acc.at[dis[b]], ssem[b], add=True)

    def wait_scatter(b):
        pltpu.make_async_copy(rbs[b], acc.at[dis[b]], ssem[b]).wait()

    issue_gather(0, 0)
    issue_gather(1, 1)
    wait_gather(0)
    compute_scale(0)
    issue_scatter(0)
    issue_gather(2, 2)

    def _steady(o, _):
        c0 = 1 + o * 3
        for bb in range(3):
            ci = c0 + bb
            b = (1 + bb) % 3
            wait_gather(b)
            compute_scale(b)
            issue_scatter(b)
            b2 = (b + 2) % 3
            wait_scatter(b2)
            issue_gather(ci + 2, b2)
        return 0
    lax.fori_loop(0, (NCH - 5) // 3, _steady, 0)

    for ci in range(NCH - 4, NCH):
        b = ci % 3
        wait_gather(b)
        compute_scale(b)
        issue_scatter(b)
        if ci + 2 < NCH:
            b2 = (b + 2) % 3
            wait_scatter(b2)
            issue_gather(ci + 2, b2)
    for ci in range(NCH - 3, NCH):
        wait_scatter(ci % 3)

    plsc.subcore_barrier()
    for k in range(CPT // K):
        pltpu.sync_copy(acc.at[pl.ds(row0 + k * K, K)],
                        out_hbm.at[cid, pl.ds(row0 + k * K, K)])


_alpha_call = pl.kernel(
    _alpha_body,
    out_type=jax.ShapeDtypeStruct((E,), jnp.float32),
    mesh=plsc.VectorSubcoreMesh(core_axis_name="c", subcore_axis_name="s"),
    compiler_params=pltpu.CompilerParams(needs_layout_passes=False),
    scratch_types=[
        pltpu.VMEM((N,), jnp.float32),
        pltpu.VMEM((N,), jnp.float32),
        pltpu.VMEM((NUM_RELS,), jnp.float32),
        pltpu.VMEM((EPS,), jnp.float32),
        pltpu.VMEM((EPS,), jnp.int32),
        pltpu.VMEM((SUB,), jnp.int32),
        pltpu.VMEM((SUB,), jnp.int32),
        pltpu.VMEM((SUB,), jnp.float32),
        pltpu.VMEM((NPAD,), jnp.float32),
        pltpu.VMEM((CPT,), jnp.float32),
        pltpu.VMEM((CPT,), jnp.float32),
        pltpu.VMEM((L,), jnp.float32),
        pltpu.VMEM((NS, L), jnp.float32),
        pltpu.VMEM_SHARED((NS, L), jnp.float32),
        pltpu.VMEM_SHARED((NS, NPAD), jnp.float32),
    ],
)

_msg_call = pl.kernel(
    _msg_body,
    out_type=jax.ShapeDtypeStruct((NC, NPAD, HID), jnp.float32),
    mesh=plsc.VectorSubcoreMesh(core_axis_name="c", subcore_axis_name="s"),
    compiler_params=pltpu.CompilerParams(needs_layout_passes=False),
    scratch_types=[
        pltpu.VMEM((K, HID), jnp.float32),
        pltpu.VMEM((K, HID), jnp.float32),
        pltpu.VMEM((K, HID), jnp.float32),
        pltpu.VMEM((K,), jnp.int32),
        pltpu.VMEM((K,), jnp.int32),
        pltpu.VMEM((K,), jnp.int32),
        pltpu.VMEM((K,), jnp.int32),
        pltpu.VMEM((K,), jnp.int32),
        pltpu.VMEM((K,), jnp.int32),
        pltpu.VMEM((K,), jnp.float32),
        pltpu.VMEM((K,), jnp.float32),
        pltpu.VMEM((K,), jnp.float32),
        pltpu.SemaphoreType.DMA,
        pltpu.SemaphoreType.DMA,
        pltpu.SemaphoreType.DMA,
        pltpu.SemaphoreType.DMA,
        pltpu.SemaphoreType.DMA,
        pltpu.SemaphoreType.DMA,
        pltpu.VMEM_SHARED((NPAD, HID), jnp.float32),
    ],
)


def kernel(x, edge_index, edge_type_in, edge_attr, W_msg, rel_emb, W_rel,
           att_vec, bias):
    src = edge_index[0]
    dst = edge_index[1]
    conf2 = edge_attr.reshape(E // HID, HID)
    h, sd2, ss2, rs2, cl2 = pl.pallas_call(
        _prep_body,
        out_shape=[
            jax.ShapeDtypeStruct((N, HID), jnp.float32),
            jax.ShapeDtypeStruct((N, 1), jnp.float32),
            jax.ShapeDtypeStruct((N, 1), jnp.float32),
            jax.ShapeDtypeStruct((NUM_RELS, 1), jnp.float32),
            jax.ShapeDtypeStruct((E // HID, HID), jnp.float32),
        ],
    )(x, W_msg, rel_emb, W_rel, att_vec, conf2)
    alpha = _alpha_call(src, dst, edge_type_in, cl2.reshape(E),
                        sd2.reshape(N), ss2.reshape(N), rs2.reshape(NUM_RELS))
    parts = _msg_call(src, dst, alpha, h)
    out = pl.pallas_call(
        _fin_body,
        out_shape=jax.ShapeDtypeStruct((N, HID), jnp.float32),
    )(parts, bias)
    return out


# speedup vs baseline: 21.5878x; 1.0854x over previous
import jax
import jax.numpy as jnp
from jax import lax
from jax.experimental import pallas as pl
from jax.experimental.pallas import tpu as pltpu
from jax.experimental.pallas import tpu_sc as plsc

N = 10000
E = 320000
HID = 128
NUM_RELS = 16
CONF_LOG_WEIGHT = 0.5

NC, NS, L = 2, 16, 16
NPAD = 10240
CPT = NPAD // NS
EPS = E // NS
SUB = 2000
NSUB = EPS // SUB
EPW = E // (NC * NS)
K = 80
NCH = EPW // K
RPB = HID // L


def _prep_body(x_ref, wm_ref, re_ref, wr_ref, av_ref, cf_ref,
               h_ref, sd_ref, ss_ref, rs_ref, cl_ref):
    x = x_ref[...]
    h = lax.dot_general(x, wm_ref[...], (((1,), (1,)), ((), ())),
                        preferred_element_type=jnp.float32)
    h_ref[...] = h
    av = av_ref[...]
    sd_ref[...] = jnp.sum(h * av[0:HID][None, :], axis=1, keepdims=True)
    ss_ref[...] = jnp.sum(h * av[HID:2 * HID][None, :], axis=1, keepdims=True)
    rp = lax.dot_general(re_ref[...], wr_ref[...], (((1,), (1,)), ((), ())),
                         preferred_element_type=jnp.float32)
    rs_ref[...] = jnp.sum(rp * av[2 * HID:][None, :], axis=1, keepdims=True)
    cl_ref[...] = CONF_LOG_WEIGHT * jnp.log(jnp.maximum(cf_ref[...], 1e-6))


def _fin_body(p_ref, b_ref, o_ref):
    o_ref[...] = p_ref[0, :N, :] + p_ref[1, :N, :] + b_ref[...][None, :]


def _alpha_body(src_hbm, dst_hbm, typ_hbm, clog_hbm, sd_hbm, ss_hbm, rsc_hbm,
                al_hbm,
                sd_v, ss_v, rsc_v, e_v, dst_v, sb, tb, cb,
                den_v, mrg_v, tsl_v, mx_v, gst_v,
                gstat, dstg):
    cid = lax.axis_index("c")
    tid = lax.axis_index("s")
    zeros16 = jnp.zeros((L,), jnp.float32)

    pltpu.sync_copy(sd_hbm, sd_v)
    pltpu.sync_copy(ss_hbm, ss_v)
    pltpu.sync_copy(rsc_hbm, rsc_v)

    def _zden(j, _):
        den_v[pl.ds(pl.multiple_of(j * L, L), L)] = zeros16
        return 0
    lax.fori_loop(0, NPAD // L, _zden, 0)

    stats_base = tid * EPS
    mx = jnp.full((L,), -1e30, jnp.float32)
    for sub in range(NSUB):
        off = stats_base + sub * SUB
        pltpu.sync_copy(src_hbm.at[pl.ds(off, SUB)], sb)
        pltpu.sync_copy(typ_hbm.at[pl.ds(off, SUB)], tb)
        pltpu.sync_copy(clog_hbm.at[pl.ds(off, SUB)], cb)
        pltpu.sync_copy(dst_hbm.at[pl.ds(off, SUB)],
                        dst_v.at[pl.ds(sub * SUB, SUB)])

        def _grp(g, mxc, sub=sub):
            o = pl.multiple_of(g * L, L)
            d16 = dst_v[pl.ds(sub * SUB + o, L)]
            s16 = sb[pl.ds(o, L)]
            t16 = tb[pl.ds(o, L)]
            c16 = cb[pl.ds(o, L)]
            sd = plsc.load_gather(sd_v, [d16])
            ssg = plsc.load_gather(ss_v, [s16])
            tt = jnp.clip(t16, 0, NUM_RELS - 1)
            rr = plsc.load_gather(rsc_v, [tt])
            a = sd + ssg + rr
            a = jnp.where(a >= 0, a, 0.2 * a)
            e16 = a + c16
            e_v[pl.ds(sub * SUB + o, L)] = e16
            return jnp.maximum(mxc, e16)
        mx = lax.fori_loop(0, SUB // L, _grp, mx)

    mx_v[...] = mx
    pltpu.sync_copy(mx_v, gstat.at[tid])
    plsc.subcore_barrier()
    pltpu.sync_copy(gstat, gst_v)
    m16 = gst_v[0, :]
    for j in range(1, NS):
        m16 = jnp.maximum(m16, gst_v[j, :])
    gmax = jnp.max(m16)

    def _dgrp(g, _):
        o = pl.multiple_of(g * L, L)
        e16 = e_v[pl.ds(o, L)]
        d16 = dst_v[pl.ds(o, L)]
        ex = jnp.exp(e16 - gmax)
        plsc.addupdate_scatter(den_v, [d16], ex)
        return 0
    lax.fori_loop(0, EPS // L, _dgrp, 0)

    pltpu.sync_copy(den_v, dstg.at[tid])
    plsc.subcore_barrier()
    colbase = tid * CPT
    pltpu.sync_copy(dstg.at[0, pl.ds(colbase, CPT)], mrg_v)
    for j in range(1, NS):
        pltpu.sync_copy(dstg.at[j, pl.ds(colbase, CPT)], tsl_v)

        def _macc(q, _):
            o = pl.multiple_of(q * L, L)
            mrg_v[pl.ds(o, L)] = mrg_v[pl.ds(o, L)] + tsl_v[pl.ds(o, L)]
            return 0
        lax.fori_loop(0, CPT // L, _macc, 0)
    pltpu.sync_copy(mrg_v, dstg.at[0, pl.ds(colbase, CPT)])
    plsc.subcore_barrier()
    pltpu.sync_copy(dstg.at[0], den_v)

    msg_base = (tid * NC + cid) * EPW
    loc = cid * EPW
    for blk in range(EPW // SUB):
        def _agrp(g, _, blk=blk):
            o = pl.multiple_of(g * L, L)
            e16 = e_v[pl.ds(loc + blk * SUB + o, L)]
            d16 = dst_v[pl.ds(loc + blk * SUB + o, L)]
            den16 = plsc.load_gather(den_v, [d16])
            cb[pl.ds(o, L)] = jnp.exp(e16 - gmax) / (den16 + 1e-16)
            return 0
        lax.fori_loop(0, SUB // L, _agrp, 0)
        pltpu.sync_copy(cb, al_hbm.at[pl.ds(msg_base + blk * SUB, SUB)])


def _msg_body(src_hbm, dst_hbm, al_hbm, h_hbm, out_hbm,
              rb0, rb1, rb2, si0, si1, si2, di0, di1, di2, ab0, ab1, ab2,
              gs0, gs1, gs2, sc0, sc1, sc2, xs0, xs1, xs2,
              acc):
    cid = lax.axis_index("c")
    tid = lax.axis_index("s")
    rbs = (rb0, rb1, rb2)
    sis = (si0, si1, si2)
    dis = (di0, di1, di2)
    abs_ = (ab0, ab1, ab2)
    gsem = (gs0, gs1, gs2)
    ssem = (sc0, sc1, sc2)
    xsem = (xs0, xs1, xs2)
    zeros16 = jnp.zeros((L,), jnp.float32)

    def _zrow(j, _):
        for r in range(RPB):
            rb0[j, pl.ds(r * L, L)] = zeros16
        return 0
    lax.fori_loop(0, K, _zrow, 0)
    row0 = tid * CPT
    for k in range(CPT // K):
        pltpu.sync_copy(rb0, acc.at[pl.ds(row0 + k * K, K)])
    plsc.subcore_barrier()

    msg_base = (tid * NC + cid) * EPW

    def issue_idx(ci, b):
        goff = pl.multiple_of(msg_base + ci * K, 8)
        pltpu.async_copy(src_hbm.at[pl.ds(goff, K)], sis[b], xsem[b])
        pltpu.async_copy(dst_hbm.at[pl.ds(goff, K)], dis[b], xsem[b])
        pltpu.async_copy(al_hbm.at[pl.ds(goff, K)], abs_[b], xsem[b])

    def issue_rows(ci, b):
        goff = pl.multiple_of(msg_base + ci * K, 8)
        pltpu.make_async_copy(src_hbm.at[pl.ds(goff, K)], sis[b],
                              xsem[b]).wait()
        pltpu.make_async_copy(dst_hbm.at[pl.ds(goff, K)], dis[b],
                              xsem[b]).wait()
        pltpu.make_async_copy(al_hbm.at[pl.ds(goff, K)], abs_[b],
                              xsem[b]).wait()
        pltpu.async_copy(h_hbm.at[sis[b]], rbs[b], gsem[b])

    def wait_rows(b):
        pltpu.make_async_copy(h_hbm.at[sis[b]], rbs[b], gsem[b]).wait()

    def compute_scale(b):
        def _srow(jj, _):
            j0 = jj * 4
            for u in range(4):
                s16 = plsc.load_gather(
                    abs_[b], [jnp.full((L,), j0 + u, jnp.int32)])
                for r in range(RPB):
                    rbs[b][j0 + u, pl.ds(r * L, L)] = (
                        rbs[b][j0 + u, pl.ds(r * L, L)] * s16)
            return 0
        lax.fori_loop(0, K // 4, _srow, 0)

    def issue_scatter(b):
        pltpu.async_copy(rbs[b], acc.at[dis[b]], ssem[b], add=True)

    def wait_scatter(b):
        pltpu.make_async_copy(rbs[b], acc.at[dis[b]], ssem[b]).wait()

    def process(ci, b, first=False, last_idx=False, last_rows=False):
        wait_rows(b)
        compute_scale(b)
        issue_scatter(b)
        b2 = (b + 2) % 3
        if not first:
            wait_scatter(b2)
        if not last_idx:
            issue_idx(ci + 2, b2)
        if not last_rows:
            issue_rows(ci + 1, (b + 1) % 3)

    issue_idx(0, 0)
    issue_rows(0, 0)
    issue_idx(1, 1)
    process(0, 0, first=True)

    def _steady(o, _):
        c0 = 1 + o * 3
        for bb in range(3):
            process(c0 + bb, (1 + bb) % 3)
        return 0
    lax.fori_loop(0, (NCH - 5) // 3, _steady, 0)

    for ci in range(NCH - 4, NCH):
        process(ci, ci % 3, last_idx=(ci + 2 >= NCH), last_rows=(ci + 1 >= NCH))
    wait_scatter((NCH - 1) % 3)

    plsc.subcore_barrier()
    for k in range(CPT // K):
        pltpu.sync_copy(acc.at[pl.ds(row0 + k * K, K)],
                        out_hbm.at[cid, pl.ds(row0 + k * K, K)])


_alpha_call = pl.kernel(
    _alpha_body,
    out_type=jax.ShapeDtypeStruct((E,), jnp.float32),
    mesh=plsc.VectorSubcoreMesh(core_axis_name="c", subcore_axis_name="s"),
    compiler_params=pltpu.CompilerParams(needs_layout_passes=False),
    scratch_types=[
        pltpu.VMEM((N,), jnp.float32),
        pltpu.VMEM((N,), jnp.float32),
        pltpu.VMEM((NUM_RELS,), jnp.float32),
        pltpu.VMEM((EPS,), jnp.float32),
        pltpu.VMEM((EPS,), jnp.int32),
        pltpu.VMEM((SUB,), jnp.int32),
        pltpu.VMEM((SUB,), jnp.int32),
        pltpu.VMEM((SUB,), jnp.float32),
        pltpu.VMEM((NPAD,), jnp.float32),
        pltpu.VMEM((CPT,), jnp.float32),
        pltpu.VMEM((CPT,), jnp.float32),
        pltpu.VMEM((L,), jnp.float32),
        pltpu.VMEM((NS, L), jnp.float32),
        pltpu.VMEM_SHARED((NS, L), jnp.float32),
        pltpu.VMEM_SHARED((NS, NPAD), jnp.float32),
    ],
)

_msg_call = pl.kernel(
    _msg_body,
    out_type=jax.ShapeDtypeStruct((NC, NPAD, HID), jnp.float32),
    mesh=plsc.VectorSubcoreMesh(core_axis_name="c", subcore_axis_name="s"),
    compiler_params=pltpu.CompilerParams(needs_layout_passes=False),
    scratch_types=[
        pltpu.VMEM((K, HID), jnp.float32),
        pltpu.VMEM((K, HID), jnp.float32),
        pltpu.VMEM((K, HID), jnp.float32),
        pltpu.VMEM((K,), jnp.int32),
        pltpu.VMEM((K,), jnp.int32),
        pltpu.VMEM((K,), jnp.int32),
        pltpu.VMEM((K,), jnp.int32),
        pltpu.VMEM((K,), jnp.int32),
        pltpu.VMEM((K,), jnp.int32),
        pltpu.VMEM((K,), jnp.float32),
        pltpu.VMEM((K,), jnp.float32),
        pltpu.VMEM((K,), jnp.float32),
        pltpu.SemaphoreType.DMA,
        pltpu.SemaphoreType.DMA,
        pltpu.SemaphoreType.DMA,
        pltpu.SemaphoreType.DMA,
        pltpu.SemaphoreType.DMA,
        pltpu.SemaphoreType.DMA,
        pltpu.SemaphoreType.DMA,
        pltpu.SemaphoreType.DMA,
        pltpu.SemaphoreType.DMA,
        pltpu.VMEM_SHARED((NPAD, HID), jnp.float32),
    ],
)


def kernel(x, edge_index, edge_type_in, edge_attr, W_msg, rel_emb, W_rel,
           att_vec, bias):
    src = edge_index[0]
    dst = edge_index[1]
    conf2 = edge_attr.reshape(E // HID, HID)
    h, sd2, ss2, rs2, cl2 = pl.pallas_call(
        _prep_body,
        out_shape=[
            jax.ShapeDtypeStruct((N, HID), jnp.float32),
            jax.ShapeDtypeStruct((N, 1), jnp.float32),
            jax.ShapeDtypeStruct((N, 1), jnp.float32),
            jax.ShapeDtypeStruct((NUM_RELS, 1), jnp.float32),
            jax.ShapeDtypeStruct((E // HID, HID), jnp.float32),
        ],
    )(x, W_msg, rel_emb, W_rel, att_vec, conf2)
    alpha = _alpha_call(src, dst, edge_type_in, cl2.reshape(E),
                        sd2.reshape(N), ss2.reshape(N), rs2.reshape(NUM_RELS))
    parts = _msg_call(src, dst, alpha, h)
    out = pl.pallas_call(
        _fin_body,
        out_shape=jax.ShapeDtypeStruct((N, HID), jnp.float32),
    )(parts, bias)
    return out
